# baseline (speedup 1.0000x reference)
import jax
import jax.numpy as jnp
from jax import lax
from jax.experimental import pallas as pl
from jax.experimental.pallas import tpu as pltpu
from jax.experimental.pallas import tpu_sc as plsc

B = 8192
D = 256
S = 10
H = 1024
E = 256

NW = 32
NPW = B // NW
NB = 32
ROUNDS = NPW // NB
NH = S // 2
UNITS = ROUNDS * 2
NODE_CH = 64
NODE_N = NPW // NODE_CH
BLK = 1024
NBLK = B // BLK


def _sc_nodes_body(nodes_hbm, table_hbm, nfeat_hbm, nodeidx_v, buf_v,
                   sem, sem2):
    cid = lax.axis_index("c")
    sid = lax.axis_index("s")
    wid = sid * 2 + cid
    base = wid * NPW
    pltpu.sync_copy(nodes_hbm.at[wid], nodeidx_v)
    cp = pltpu.async_copy(table_hbm.at[nodeidx_v.at[0]], buf_v.at[0], sem)
    for j in range(NPW // 128):
        cp.wait()
        if j + 1 < NPW // 128:
            cp = pltpu.async_copy(table_hbm.at[nodeidx_v.at[j + 1]],
                                  buf_v.at[(j + 1) % 2],
                                  sem if (j + 1) % 2 == 0 else sem2)
        pltpu.sync_copy(buf_v.at[j % 2],
                        nfeat_hbm.at[pl.ds(base + j * 128, 128)])


@jax.jit
def _sc_nodes(nodes2d, table):
    mesh = plsc.VectorSubcoreMesh(core_axis_name="c", subcore_axis_name="s")
    f = pl.kernel(
        _sc_nodes_body,
        out_type=jax.ShapeDtypeStruct((B, D), jnp.float32),
        mesh=mesh,
        scratch_types=[
            pltpu.VMEM((NPW // 128, 128), jnp.int32),
            pltpu.VMEM((2, 128, D), jnp.float32),
            pltpu.SemaphoreType.DMA,
            pltpu.SemaphoreType.DMA,
        ],
    )
    return f(nodes2d, table)


def _sc_mean_body(nidx_hbm, table_hbm, nmean_hbm,
                  nidx_v, bufs_v, accs_v, semA, semB, semM):
    cid = lax.axis_index("c")
    sid = lax.axis_index("s")
    wid = sid * 2 + cid
    base = wid * NPW

    pltpu.sync_copy(nidx_hbm.at[wid], nidx_v)

    def fire_unit(u):
        p = u % 2
        sm = semA if p == 0 else semB
        return [pltpu.async_copy(table_hbm.at[nidx_v.at[u * NH + so]],
                                 bufs_v.at[p, so], sm)
                for so in range(NH)]

    cps = fire_unit(0)
    acc_w = {}
    for u in range(UNITS):
        r, q = divmod(u, 2)
        accp = r % 2
        if q == 0 and r >= 2:
            acc_w.pop(accp).wait()
        for c in cps:
            c.wait()
        if u + 1 < UNITS:
            cps = fire_unit(u + 1)

        def nbody(n, carry):
            for d in range(D // 16):
                sl = pl.ds(d * 16, 16)
                a = bufs_v[q, 0, n, sl]
                for so in range(1, NH):
                    a = a + bufs_v[q, so, n, sl]
                if q == 0:
                    accs_v[accp, n, sl] = a
                else:
                    accs_v[accp, n, sl] = (
                        (accs_v[accp, n, sl] + a) * jnp.float32(1.0 / S))
            return carry

        lax.fori_loop(0, NB, nbody, 0)

        if q == 1:
            acc_w[accp] = pltpu.async_copy(
                accs_v.at[accp], nmean_hbm.at[pl.ds(base + r * NB, NB)],
                semM)

    for c in acc_w.values():
        c.wait()


@jax.jit
def _sc_mean(nidx2d, table):
    mesh = plsc.VectorSubcoreMesh(core_axis_name="c", subcore_axis_name="s")
    f = pl.kernel(
        _sc_mean_body,
        out_type=jax.ShapeDtypeStruct((B, D), jnp.float32),
        mesh=mesh,
        scratch_types=[
            pltpu.VMEM((UNITS * NH, NB), jnp.int32),
            pltpu.VMEM((2, NH, NB, D), jnp.float32),
            pltpu.VMEM((2, NB, D), jnp.float32),
            pltpu.SemaphoreType.DMA,
            pltpu.SemaphoreType.DMA,
            pltpu.SemaphoreType.DMA,
        ],
    )
    return f(nidx2d, table)


def _usum_body(x_ref, w2_ref, zt_ref, out_ref, acc_ref):
    i = pl.program_id(0)

    @pl.when(i == 0)
    def _init():
        acc_ref[0] = jnp.float32(0.0)

    t = jnp.tanh(jnp.dot(x_ref[...].astype(jnp.bfloat16),
                         w2_ref[...].astype(jnp.bfloat16),
                         preferred_element_type=jnp.float32))
    acc_ref[0] += jnp.sum(t * zt_ref[...])

    @pl.when(i == NBLK - 1)
    def _fin():
        out_ref[0] = acc_ref[0] / B


@jax.jit
def _usum(x, w2, zt):
    return pl.pallas_call(
        _usum_body,
        grid=(NBLK,),
        in_specs=[
            pl.BlockSpec((BLK, D), lambda i: (i, 0)),
            pl.BlockSpec((D, H), lambda i: (0, 0)),
            pl.BlockSpec((1, H), lambda i: (0, 0)),
        ],
        out_specs=pl.BlockSpec(memory_space=pltpu.SMEM),
        out_shape=jax.ShapeDtypeStruct((1,), jnp.float32),
        scratch_shapes=[pltpu.SMEM((1,), jnp.float32)],
    )(x, w2, zt)


def _combine_body(u_ref, nfeat_ref, nmean_ref, w_ref, out_ref):
    u0 = u_ref[0]
    u1 = u_ref[1]
    m = jnp.maximum(u0, u1)
    e0 = jnp.exp(u0 - m)
    e1 = jnp.exp(u1 - m)
    a0 = e0 / (e0 + e1)
    a1 = e1 / (e0 + e1)
    comb = jnp.maximum(a0 * nfeat_ref[...] + a1 * nmean_ref[...], 0.0)
    out_ref[...] = jnp.maximum(
        lax.dot_general(w_ref[...], comb, (((1,), (1,)), ((), ())),
                        preferred_element_type=jnp.float32),
        0.0)


@jax.jit
def _combine(u, nfeat, nmean, w):
    return pl.pallas_call(
        _combine_body,
        grid=(NBLK,),
        in_specs=[
            pl.BlockSpec(memory_space=pltpu.SMEM),
            pl.BlockSpec((BLK, D), lambda i: (i, 0)),
            pl.BlockSpec((BLK, D), lambda i: (i, 0)),
            pl.BlockSpec((E, D), lambda i: (0, 0)),
        ],
        out_specs=pl.BlockSpec((E, BLK), lambda i: (0, i)),
        out_shape=jax.ShapeDtypeStruct((E, B), jnp.float32),
    )(u, nfeat, nmean, w)


def kernel(nodes, neigh_idx, features_table, weight, weight_2, z):
    nodes2d = nodes.astype(jnp.int32).reshape(NW, NPW // 128, 128)
    nidx2d = (neigh_idx.astype(jnp.int32)
              .reshape(NW, ROUNDS, NB, 2, NH)
              .transpose(0, 1, 3, 4, 2)
              .reshape(NW, UNITS * NH, NB))
    zt = z.reshape(1, H)
    nfeat = _sc_nodes(nodes2d, features_table)
    nmean = _sc_mean(nidx2d, features_table)
    u0 = _usum(nfeat, weight_2, zt)
    u1 = _usum(nmean, weight_2, zt)
    u = jnp.concatenate([u0, u1])
    return _combine(u, nfeat, nmean, weight)


# speedup vs baseline: 1.0565x; 1.0565x over previous
import jax
import jax.numpy as jnp
from jax import lax
from jax.experimental import pallas as pl
from jax.experimental.pallas import tpu as pltpu
from jax.experimental.pallas import tpu_sc as plsc

B = 8192
D = 256
S = 10
H = 1024
E = 256

NW = 32
NPW = B // NW
NB = 32
ROUNDS = NPW // NB
NH = S // 2
UNITS = ROUNDS * 2
NODE_CH = 64
NODE_N = NPW // NODE_CH
BLK = 1024
NBLK = B // BLK


def _sc_body(nodes_hbm, nidx_hbm, table_hbm, nfeat_hbm, nmean_hbm,
             nidx_v, nodeidx_v, bufs_v, accs_v, nodebuf_v,
             semA, semB, semN, semN2, semM):
    cid = lax.axis_index("c")
    sid = lax.axis_index("s")
    wid = sid * 2 + cid
    base = wid * NPW

    pltpu.sync_copy(nodes_hbm.at[wid], nodeidx_v)
    pltpu.sync_copy(nidx_hbm.at[wid], nidx_v)

    def fire_unit(u):
        p = u % 2
        sm = semA if p == 0 else semB
        return [pltpu.async_copy(table_hbm.at[nidx_v.at[u * NH + so]],
                                 bufs_v.at[p, so], sm)
                for so in range(NH)]

    ncp = pltpu.async_copy(table_hbm.at[nodeidx_v.at[0]], nodebuf_v, semN)
    node_w = []
    cps = fire_unit(0)
    acc_w = {}
    for u in range(UNITS):
        r, q = divmod(u, 2)
        accp = r % 2
        if q == 0 and r >= 2:
            acc_w.pop(accp).wait()
        for c in cps:
            c.wait()
        if u + 1 < UNITS:
            cps = fire_unit(u + 1)

        def nbody(n, carry):
            for d in range(D // 16):
                sl = pl.ds(d * 16, 16)
                a = bufs_v[q, 0, n, sl]
                for so in range(1, NH):
                    a = a + bufs_v[q, so, n, sl]
                if q == 0:
                    accs_v[accp, n, sl] = a
                else:
                    accs_v[accp, n, sl] = (
                        (accs_v[accp, n, sl] + a) * jnp.float32(1.0 / S))
            return carry

        lax.fori_loop(0, NB, nbody, 0)

        if q == 1:
            acc_w[accp] = pltpu.async_copy(
                accs_v.at[accp], nmean_hbm.at[pl.ds(base + r * NB, NB)],
                semM)

        if u % 2 == 1 and u // 2 < NODE_N:
            k = u // 2
            ncp.wait()
            node_w.append(pltpu.async_copy(
                nodebuf_v, nfeat_hbm.at[pl.ds(base + k * NODE_CH, NODE_CH)],
                semN2))
            if k + 1 < NODE_N:
                node_w[-1].wait()
                ncp = pltpu.async_copy(table_hbm.at[nodeidx_v.at[k + 1]],
                                       nodebuf_v, semN)

    for c in acc_w.values():
        c.wait()
    node_w[-1].wait()


@jax.jit
def _sc_gather(nodes2d, nidx2d, table):
    mesh = plsc.VectorSubcoreMesh(core_axis_name="c", subcore_axis_name="s")
    f = pl.kernel(
        _sc_body,
        out_type=(
            jax.ShapeDtypeStruct((B, D), jnp.float32),
            jax.ShapeDtypeStruct((B, D), jnp.float32),
        ),
        mesh=mesh,
        scratch_types=[
            pltpu.VMEM((UNITS * NH, NB), jnp.int32),
            pltpu.VMEM((NODE_N, NODE_CH), jnp.int32),
            pltpu.VMEM((2, NH, NB, D), jnp.float32),
            pltpu.VMEM((2, NB, D), jnp.float32),
            pltpu.VMEM((NODE_CH, D), jnp.float32),
            pltpu.SemaphoreType.DMA,
            pltpu.SemaphoreType.DMA,
            pltpu.SemaphoreType.DMA,
            pltpu.SemaphoreType.DMA,
            pltpu.SemaphoreType.DMA,
        ],
    )
    return f(nodes2d, nidx2d, table)


def _tc_body(nfeat_ref, nmean_ref, w2_ref, z_ref, w_ref, out_ref, acc_ref):
    p = pl.program_id(0)

    @pl.when((p == 0) & (pl.program_id(1) == 0))
    def _init():
        acc_ref[0] = jnp.float32(0.0)
        acc_ref[1] = jnp.float32(0.0)

    @pl.when(p == 0)
    def _sums():
        w2b = w2_ref[...].astype(jnp.bfloat16)
        zc = z_ref[...]
        tn = jnp.tanh(jnp.dot(nfeat_ref[...].astype(jnp.bfloat16), w2b,
                              preferred_element_type=jnp.float32))
        tm = jnp.tanh(jnp.dot(nmean_ref[...].astype(jnp.bfloat16), w2b,
                              preferred_element_type=jnp.float32))
        acc_ref[0] += jnp.sum(jnp.dot(tn, zc,
                                      preferred_element_type=jnp.float32))
        acc_ref[1] += jnp.sum(jnp.dot(tm, zc,
                                      preferred_element_type=jnp.float32))

    @pl.when(p == 1)
    def _combine():
        u0 = acc_ref[0] / B
        u1 = acc_ref[1] / B
        m = jnp.maximum(u0, u1)
        e0 = jnp.exp(u0 - m)
        e1 = jnp.exp(u1 - m)
        a0 = e0 / (e0 + e1)
        a1 = e1 / (e0 + e1)
        comb = jnp.maximum(a0 * nfeat_ref[...] + a1 * nmean_ref[...], 0.0)
        out_ref[...] = jnp.maximum(
            lax.dot_general(w_ref[...], comb, (((1,), (1,)), ((), ())),
                            preferred_element_type=jnp.float32),
            0.0)


@jax.jit
def _tc_fused(nfeat, nmean, w2, z, w):
    return pl.pallas_call(
        _tc_body,
        grid=(2, NBLK),
        in_specs=[
            pl.BlockSpec((BLK, D), lambda p, i: (i, 0)),
            pl.BlockSpec((BLK, D), lambda p, i: (i, 0)),
            pl.BlockSpec((D, H), lambda p, i: (0, 0)),
            pl.BlockSpec((H, 1), lambda p, i: (0, 0)),
            pl.BlockSpec((E, D), lambda p, i: (0, 0)),
        ],
        out_specs=pl.BlockSpec((E, BLK),
                               lambda p, i: (0, jnp.where(p == 0, 0, i))),
        out_shape=jax.ShapeDtypeStruct((E, B), jnp.float32),
        scratch_shapes=[
            pltpu.SMEM((2,), jnp.float32),
        ],
    )(nfeat, nmean, w2, z, w)


def kernel(nodes, neigh_idx, features_table, weight, weight_2, z):
    nodes2d = nodes.astype(jnp.int32).reshape(NW, NODE_N, NODE_CH)
    nidx2d = (neigh_idx.astype(jnp.int32)
              .reshape(NW, ROUNDS, NB, 2, NH)
              .transpose(0, 1, 3, 4, 2)
              .reshape(NW, UNITS * NH, NB))
    nfeat, nmean = _sc_gather(nodes2d, nidx2d, features_table)
    return _tc_fused(nfeat, nmean, weight_2, z, weight)


# speedup vs baseline: 1.0810x; 1.0232x over previous
import jax
import jax.numpy as jnp
from jax import lax
from jax.experimental import pallas as pl
from jax.experimental.pallas import tpu as pltpu
from jax.experimental.pallas import tpu_sc as plsc

B = 8192
D = 256
S = 10
H = 1024
E = 256

NW = 32
NPW = B // NW
NB = 32
ROUNDS = NPW // NB
NH = S // 2
UNITS = ROUNDS * 2
NODE_CH = 64
NODE_N = NPW // NODE_CH
BLK = 1024
NBLK = B // BLK


def _sc_body(nodes_hbm, nidx_hbm, table_hbm, nfeat_hbm, nmean_hbm,
             nidx_v, nodeidx_v, bufs_v, accs_v, nodebuf_v,
             semA, semB, semN, semN2, semM):
    cid = lax.axis_index("c")
    sid = lax.axis_index("s")
    wid = sid * 2 + cid
    base = wid * NPW

    pltpu.sync_copy(nodes_hbm.at[wid], nodeidx_v)
    pltpu.sync_copy(nidx_hbm.at[wid], nidx_v)

    def fire_unit(u):
        p = u % 2
        sm = semA if p == 0 else semB
        return [pltpu.async_copy(table_hbm.at[nidx_v.at[u * NH + so]],
                                 bufs_v.at[p, so], sm)
                for so in range(NH)]

    ncp = pltpu.async_copy(table_hbm.at[nodeidx_v.at[0]], nodebuf_v, semN)
    node_w = []
    cps = fire_unit(0)
    acc_w = {}
    for u in range(UNITS):
        r, q = divmod(u, 2)
        accp = r % 2
        if q == 0 and r >= 2:
            acc_w.pop(accp).wait()
        for c in cps:
            c.wait()
        if u + 1 < UNITS:
            cps = fire_unit(u + 1)

        def nbody(n, carry):
            for d in range(D // 16):
                sl = pl.ds(d * 16, 16)
                a = bufs_v[q, 0, n, sl]
                for so in range(1, NH):
                    a = a + bufs_v[q, so, n, sl]
                if q == 0:
                    accs_v[accp, n, sl] = a
                else:
                    accs_v[accp, n, sl] = (
                        (accs_v[accp, n, sl] + a) * jnp.float32(1.0 / S))
            return carry

        lax.fori_loop(0, NB, nbody, 0)

        if q == 1:
            acc_w[accp] = pltpu.async_copy(
                accs_v.at[accp], nmean_hbm.at[pl.ds(base + r * NB, NB)],
                semM)

        if u % 2 == 1 and u // 2 < NODE_N:
            k = u // 2
            ncp.wait()
            node_w.append(pltpu.async_copy(
                nodebuf_v, nfeat_hbm.at[pl.ds(base + k * NODE_CH, NODE_CH)],
                semN2))
            if k + 1 < NODE_N:
                node_w[-1].wait()
                ncp = pltpu.async_copy(table_hbm.at[nodeidx_v.at[k + 1]],
                                       nodebuf_v, semN)

    for c in acc_w.values():
        c.wait()
    node_w[-1].wait()


@jax.jit
def _sc_gather(nodes2d, nidx2d, table):
    mesh = plsc.VectorSubcoreMesh(core_axis_name="c", subcore_axis_name="s")
    f = pl.kernel(
        _sc_body,
        out_type=(
            jax.ShapeDtypeStruct((B, D), jnp.float32),
            jax.ShapeDtypeStruct((B, D), jnp.float32),
        ),
        mesh=mesh,
        scratch_types=[
            pltpu.VMEM((UNITS * NH, NB), jnp.int32),
            pltpu.VMEM((NODE_N, NODE_CH), jnp.int32),
            pltpu.VMEM((2, NH, NB, D), jnp.float32),
            pltpu.VMEM((2, NB, D), jnp.float32),
            pltpu.VMEM((NODE_CH, D), jnp.float32),
            pltpu.SemaphoreType.DMA,
            pltpu.SemaphoreType.DMA,
            pltpu.SemaphoreType.DMA,
            pltpu.SemaphoreType.DMA,
            pltpu.SemaphoreType.DMA,
        ],
    )
    return f(nodes2d, nidx2d, table)


def _tc_body(nfeat_ref, nmean_ref, w2_ref, zt_ref, w_ref, out_ref, acc_ref):
    p = pl.program_id(0)

    @pl.when((p == 0) & (pl.program_id(1) == 0))
    def _init():
        acc_ref[0] = jnp.float32(0.0)
        acc_ref[1] = jnp.float32(0.0)

    @pl.when(p == 0)
    def _sums():
        w2b = w2_ref[...].astype(jnp.bfloat16)
        zt = zt_ref[...]
        tn = jnp.tanh(jnp.dot(nfeat_ref[...].astype(jnp.bfloat16), w2b,
                              preferred_element_type=jnp.float32))
        tm = jnp.tanh(jnp.dot(nmean_ref[...].astype(jnp.bfloat16), w2b,
                              preferred_element_type=jnp.float32))
        acc_ref[0] += jnp.sum(tn * zt)
        acc_ref[1] += jnp.sum(tm * zt)

    @pl.when(p == 1)
    def _combine():
        u0 = acc_ref[0] / B
        u1 = acc_ref[1] / B
        m = jnp.maximum(u0, u1)
        e0 = jnp.exp(u0 - m)
        e1 = jnp.exp(u1 - m)
        a0 = e0 / (e0 + e1)
        a1 = e1 / (e0 + e1)
        comb = jnp.maximum(a0 * nfeat_ref[...] + a1 * nmean_ref[...], 0.0)
        out_ref[...] = jnp.maximum(
            lax.dot_general(w_ref[...], comb, (((1,), (1,)), ((), ())),
                            preferred_element_type=jnp.float32),
            0.0)


@jax.jit
def _tc_fused(nfeat, nmean, w2, zt, w):
    return pl.pallas_call(
        _tc_body,
        grid=(2, NBLK),
        in_specs=[
            pl.BlockSpec((BLK, D), lambda p, i: (i, 0)),
            pl.BlockSpec((BLK, D), lambda p, i: (i, 0)),
            pl.BlockSpec((D, H), lambda p, i: (0, 0)),
            pl.BlockSpec((1, H), lambda p, i: (0, 0)),
            pl.BlockSpec((E, D), lambda p, i: (0, 0)),
        ],
        out_specs=pl.BlockSpec((E, BLK),
                               lambda p, i: (0, jnp.where(p == 0, 0, i))),
        out_shape=jax.ShapeDtypeStruct((E, B), jnp.float32),
        scratch_shapes=[
            pltpu.SMEM((2,), jnp.float32),
        ],
    )(nfeat, nmean, w2, zt, w)


def kernel(nodes, neigh_idx, features_table, weight, weight_2, z):
    nodes2d = nodes.astype(jnp.int32).reshape(NW, NODE_N, NODE_CH)
    nidx2d = (neigh_idx.astype(jnp.int32)
              .reshape(NW, ROUNDS, NB, 2, NH)
              .transpose(0, 1, 3, 4, 2)
              .reshape(NW, UNITS * NH, NB))
    nfeat, nmean = _sc_gather(nodes2d, nidx2d, features_table)
    return _tc_fused(nfeat, nmean, weight_2, z.reshape(1, H), weight)


# speedup vs baseline: 1.1133x; 1.0298x over previous
import jax
import jax.numpy as jnp
from jax import lax
from jax.experimental import pallas as pl
from jax.experimental.pallas import tpu as pltpu
from jax.experimental.pallas import tpu_sc as plsc

B = 8192
D = 256
S = 10
H = 1024
E = 256

NW = 32
NPW = B // NW
NB = 32
ROUNDS = NPW // NB
NH = S // 2
UNITS = ROUNDS * 2
NODE_CH = 64
NODE_N = NPW // NODE_CH
BLK = 2048
NBLK = B // BLK


def _sc_body(nodes_hbm, nidx_hbm, table_hbm, nfeat_hbm, nmean_hbm,
             nidx_v, nodeidx_v, bufs_v, accs_v, nodebuf_v,
             semA, semB, semN, semN2, semM):
    cid = lax.axis_index("c")
    sid = lax.axis_index("s")
    wid = sid * 2 + cid
    base = wid * NPW

    pltpu.sync_copy(nodes_hbm.at[wid], nodeidx_v)
    pltpu.sync_copy(nidx_hbm.at[wid], nidx_v)

    def fire_unit(u):
        p = u % 2
        sm = semA if p == 0 else semB
        return [pltpu.async_copy(table_hbm.at[nidx_v.at[u * NH + so]],
                                 bufs_v.at[p, so], sm)
                for so in range(NH)]

    ncp = pltpu.async_copy(table_hbm.at[nodeidx_v.at[0]], nodebuf_v, semN)
    node_w = []
    cps = fire_unit(0)
    acc_w = {}
    for u in range(UNITS):
        r, q = divmod(u, 2)
        accp = r % 2
        if q == 0 and r >= 2:
            acc_w.pop(accp).wait()
        for c in cps:
            c.wait()
        if u + 1 < UNITS:
            cps = fire_unit(u + 1)

        def nbody(n, carry):
            for d in range(D // 16):
                sl = pl.ds(d * 16, 16)
                a = bufs_v[q, 0, n, sl]
                for so in range(1, NH):
                    a = a + bufs_v[q, so, n, sl]
                if q == 0:
                    accs_v[accp, n, sl] = a
                else:
                    accs_v[accp, n, sl] = (
                        (accs_v[accp, n, sl] + a) * jnp.float32(1.0 / S))
            return carry

        lax.fori_loop(0, NB, nbody, 0)

        if q == 1:
            acc_w[accp] = pltpu.async_copy(
                accs_v.at[accp], nmean_hbm.at[pl.ds(base + r * NB, NB)],
                semM)

        if u % 2 == 1 and u // 2 < NODE_N:
            k = u // 2
            ncp.wait()
            node_w.append(pltpu.async_copy(
                nodebuf_v, nfeat_hbm.at[pl.ds(base + k * NODE_CH, NODE_CH)],
                semN2))
            if k + 1 < NODE_N:
                node_w[-1].wait()
                ncp = pltpu.async_copy(table_hbm.at[nodeidx_v.at[k + 1]],
                                       nodebuf_v, semN)

    for c in acc_w.values():
        c.wait()
    node_w[-1].wait()


@jax.jit
def _sc_gather(nodes2d, nidx2d, table):
    mesh = plsc.VectorSubcoreMesh(core_axis_name="c", subcore_axis_name="s")
    f = pl.kernel(
        _sc_body,
        out_type=(
            jax.ShapeDtypeStruct((B, D), jnp.float32),
            jax.ShapeDtypeStruct((B, D), jnp.float32),
        ),
        mesh=mesh,
        scratch_types=[
            pltpu.VMEM((UNITS * NH, NB), jnp.int32),
            pltpu.VMEM((NODE_N, NODE_CH), jnp.int32),
            pltpu.VMEM((2, NH, NB, D), jnp.float32),
            pltpu.VMEM((2, NB, D), jnp.float32),
            pltpu.VMEM((NODE_CH, D), jnp.float32),
            pltpu.SemaphoreType.DMA,
            pltpu.SemaphoreType.DMA,
            pltpu.SemaphoreType.DMA,
            pltpu.SemaphoreType.DMA,
            pltpu.SemaphoreType.DMA,
        ],
    )
    return f(nodes2d, nidx2d, table)


def _tc_body(nfeat_ref, nmean_ref, w2_ref, zt_ref, w_ref, out_ref, acc_ref):
    p = pl.program_id(0)

    @pl.when((p == 0) & (pl.program_id(1) == 0))
    def _init():
        acc_ref[0] = jnp.float32(0.0)
        acc_ref[1] = jnp.float32(0.0)

    @pl.when(p == 0)
    def _sums():
        w2b = w2_ref[...].astype(jnp.bfloat16)
        zt = zt_ref[...]
        tn = jnp.tanh(jnp.dot(nfeat_ref[...].astype(jnp.bfloat16), w2b,
                              preferred_element_type=jnp.float32))
        tm = jnp.tanh(jnp.dot(nmean_ref[...].astype(jnp.bfloat16), w2b,
                              preferred_element_type=jnp.float32))
        acc_ref[0] += jnp.sum(tn * zt)
        acc_ref[1] += jnp.sum(tm * zt)

    @pl.when(p == 1)
    def _combine():
        u0 = acc_ref[0] / B
        u1 = acc_ref[1] / B
        m = jnp.maximum(u0, u1)
        e0 = jnp.exp(u0 - m)
        e1 = jnp.exp(u1 - m)
        a0 = e0 / (e0 + e1)
        a1 = e1 / (e0 + e1)
        comb = jnp.maximum(a0 * nfeat_ref[...] + a1 * nmean_ref[...], 0.0)
        out_ref[...] = jnp.maximum(
            lax.dot_general(w_ref[...], comb, (((1,), (1,)), ((), ())),
                            preferred_element_type=jnp.float32),
            0.0)


@jax.jit
def _tc_fused(nfeat, nmean, w2, zt, w):
    return pl.pallas_call(
        _tc_body,
        grid=(2, NBLK),
        in_specs=[
            pl.BlockSpec((BLK, D), lambda p, i: (i, 0)),
            pl.BlockSpec((BLK, D), lambda p, i: (i, 0)),
            pl.BlockSpec((D, H), lambda p, i: (0, 0)),
            pl.BlockSpec((1, H), lambda p, i: (0, 0)),
            pl.BlockSpec((E, D), lambda p, i: (0, 0)),
        ],
        out_specs=pl.BlockSpec((E, BLK),
                               lambda p, i: (0, jnp.where(p == 0, 0, i))),
        out_shape=jax.ShapeDtypeStruct((E, B), jnp.float32),
        scratch_shapes=[
            pltpu.SMEM((2,), jnp.float32),
        ],
    )(nfeat, nmean, w2, zt, w)


def kernel(nodes, neigh_idx, features_table, weight, weight_2, z):
    nodes2d = nodes.astype(jnp.int32).reshape(NW, NODE_N, NODE_CH)
    nidx2d = (neigh_idx.astype(jnp.int32)
              .reshape(NW, ROUNDS, NB, 2, NH)
              .transpose(0, 1, 3, 4, 2)
              .reshape(NW, UNITS * NH, NB))
    nfeat, nmean = _sc_gather(nodes2d, nidx2d, features_table)
    return _tc_fused(nfeat, nmean, weight_2, z.reshape(1, H), weight)


# speedup vs baseline: 1.1183x; 1.0045x over previous
import jax
import jax.numpy as jnp
from jax import lax
from jax.experimental import pallas as pl
from jax.experimental.pallas import tpu as pltpu
from jax.experimental.pallas import tpu_sc as plsc

B = 8192
D = 256
S = 10
H = 1024
E = 256

NW = 32
NPW = B // NW
NB = 32
ROUNDS = NPW // NB
NH = S // 2
UNITS = ROUNDS * 2
NODE_CH = 64
NODE_N = NPW // NODE_CH
BLK = 4096
NBLK = B // BLK


def _sc_body(nodes_hbm, nidx_hbm, table_hbm, nfeat_hbm, nmean_hbm,
             nidx_v, nodeidx_v, bufs_v, accs_v, nodebuf_v,
             semA, semB, semN, semN2, semM):
    cid = lax.axis_index("c")
    sid = lax.axis_index("s")
    wid = sid * 2 + cid
    base = wid * NPW

    pltpu.sync_copy(nodes_hbm.at[wid], nodeidx_v)
    pltpu.sync_copy(nidx_hbm.at[wid], nidx_v)

    def fire_unit(u):
        p = u % 2
        sm = semA if p == 0 else semB
        return [pltpu.async_copy(table_hbm.at[nidx_v.at[u * NH + so]],
                                 bufs_v.at[p, so], sm)
                for so in range(NH)]

    ncp = pltpu.async_copy(table_hbm.at[nodeidx_v.at[0]], nodebuf_v, semN)
    node_w = []
    cps = fire_unit(0)
    acc_w = {}
    for u in range(UNITS):
        r, q = divmod(u, 2)
        accp = r % 2
        if q == 0 and r >= 2:
            acc_w.pop(accp).wait()
        for c in cps:
            c.wait()
        if u + 1 < UNITS:
            cps = fire_unit(u + 1)

        def nbody(n, carry):
            for d in range(D // 16):
                sl = pl.ds(d * 16, 16)
                a = bufs_v[q, 0, n, sl]
                for so in range(1, NH):
                    a = a + bufs_v[q, so, n, sl]
                if q == 0:
                    accs_v[accp, n, sl] = a
                else:
                    accs_v[accp, n, sl] = (
                        (accs_v[accp, n, sl] + a) * jnp.float32(1.0 / S))
            return carry

        lax.fori_loop(0, NB, nbody, 0)

        if q == 1:
            acc_w[accp] = pltpu.async_copy(
                accs_v.at[accp], nmean_hbm.at[pl.ds(base + r * NB, NB)],
                semM)

        if u % 2 == 1 and u // 2 < NODE_N:
            k = u // 2
            ncp.wait()
            node_w.append(pltpu.async_copy(
                nodebuf_v, nfeat_hbm.at[pl.ds(base + k * NODE_CH, NODE_CH)],
                semN2))
            if k + 1 < NODE_N:
                node_w[-1].wait()
                ncp = pltpu.async_copy(table_hbm.at[nodeidx_v.at[k + 1]],
                                       nodebuf_v, semN)

    for c in acc_w.values():
        c.wait()
    node_w[-1].wait()


@jax.jit
def _sc_gather(nodes2d, nidx2d, table):
    mesh = plsc.VectorSubcoreMesh(core_axis_name="c", subcore_axis_name="s")
    f = pl.kernel(
        _sc_body,
        out_type=(
            jax.ShapeDtypeStruct((B, D), jnp.float32),
            jax.ShapeDtypeStruct((B, D), jnp.float32),
        ),
        mesh=mesh,
        scratch_types=[
            pltpu.VMEM((UNITS * NH, NB), jnp.int32),
            pltpu.VMEM((NODE_N, NODE_CH), jnp.int32),
            pltpu.VMEM((2, NH, NB, D), jnp.float32),
            pltpu.VMEM((2, NB, D), jnp.float32),
            pltpu.VMEM((NODE_CH, D), jnp.float32),
            pltpu.SemaphoreType.DMA,
            pltpu.SemaphoreType.DMA,
            pltpu.SemaphoreType.DMA,
            pltpu.SemaphoreType.DMA,
            pltpu.SemaphoreType.DMA,
        ],
    )
    return f(nodes2d, nidx2d, table)


def _tc_body(nfeat_ref, nmean_ref, w2_ref, zt_ref, w_ref, out_ref, acc_ref):
    p = pl.program_id(0)

    @pl.when((p == 0) & (pl.program_id(1) == 0))
    def _init():
        acc_ref[0] = jnp.float32(0.0)
        acc_ref[1] = jnp.float32(0.0)

    @pl.when(p == 0)
    def _sums():
        w2b = w2_ref[...].astype(jnp.bfloat16)
        zt = zt_ref[...]
        tn = jnp.tanh(jnp.dot(nfeat_ref[...].astype(jnp.bfloat16), w2b,
                              preferred_element_type=jnp.float32))
        tm = jnp.tanh(jnp.dot(nmean_ref[...].astype(jnp.bfloat16), w2b,
                              preferred_element_type=jnp.float32))
        acc_ref[0] += jnp.sum(tn * zt)
        acc_ref[1] += jnp.sum(tm * zt)

    @pl.when(p == 1)
    def _combine():
        u0 = acc_ref[0] / B
        u1 = acc_ref[1] / B
        m = jnp.maximum(u0, u1)
        e0 = jnp.exp(u0 - m)
        e1 = jnp.exp(u1 - m)
        a0 = e0 / (e0 + e1)
        a1 = e1 / (e0 + e1)
        comb = jnp.maximum(a0 * nfeat_ref[...] + a1 * nmean_ref[...], 0.0)
        out_ref[...] = jnp.maximum(
            lax.dot_general(w_ref[...], comb, (((1,), (1,)), ((), ())),
                            preferred_element_type=jnp.float32),
            0.0)


@jax.jit
def _tc_fused(nfeat, nmean, w2, zt, w):
    return pl.pallas_call(
        _tc_body,
        grid=(2, NBLK),
        in_specs=[
            pl.BlockSpec((BLK, D), lambda p, i: (i, 0)),
            pl.BlockSpec((BLK, D), lambda p, i: (i, 0)),
            pl.BlockSpec((D, H), lambda p, i: (0, 0)),
            pl.BlockSpec((1, H), lambda p, i: (0, 0)),
            pl.BlockSpec((E, D), lambda p, i: (0, 0)),
        ],
        out_specs=pl.BlockSpec((E, BLK),
                               lambda p, i: (0, jnp.where(p == 0, 0, i))),
        out_shape=jax.ShapeDtypeStruct((E, B), jnp.float32),
        scratch_shapes=[
            pltpu.SMEM((2,), jnp.float32),
        ],
    )(nfeat, nmean, w2, zt, w)


def kernel(nodes, neigh_idx, features_table, weight, weight_2, z):
    nodes2d = nodes.astype(jnp.int32).reshape(NW, NODE_N, NODE_CH)
    nidx2d = (neigh_idx.astype(jnp.int32)
              .reshape(NW, ROUNDS, NB, 2, NH)
              .transpose(0, 1, 3, 4, 2)
              .reshape(NW, UNITS * NH, NB))
    nfeat, nmean = _sc_gather(nodes2d, nidx2d, features_table)
    return _tc_fused(nfeat, nmean, weight_2, z.reshape(1, H), weight)
